# baseline (speedup 1.0000x reference)
import jax
import jax.numpy as jnp
from jax import lax
from jax.experimental import pallas as pl

N = 50000
C = 128
NCOEF = 25
NA = 100
BT = 400


def _emb_fill(idx_ref, table_ref, o_ref):
    onehot = (idx_ref[...] == lax.broadcasted_iota(jnp.int32, (BT, NA), 1))
    g = lax.dot(onehot.astype(jnp.float32), table_ref[...],
                precision=lax.Precision.HIGHEST,
                preferred_element_type=jnp.float32)
    o_ref[:, 0:C] = g
    o_ref[:, C:] = jnp.zeros((BT, (NCOEF - 1) * C), jnp.float32)


def kernel(atomic_numbers, embedding_table):
    idx = atomic_numbers.astype(jnp.int32).reshape(N, 1)
    out2d = pl.pallas_call(
        _emb_fill,
        grid=(N // BT,),
        in_specs=[
            pl.BlockSpec((BT, 1), lambda i: (i, 0)),
            pl.BlockSpec((NA, C), lambda i: (0, 0)),
        ],
        out_specs=pl.BlockSpec((BT, NCOEF * C), lambda i: (i, 0)),
        out_shape=jax.ShapeDtypeStruct((N, NCOEF * C), jnp.float32),
    )(idx, embedding_table)
    return out2d.reshape(N, NCOEF, C)


# speedup vs baseline: 1.5991x; 1.5991x over previous
import jax
import jax.numpy as jnp
from jax import lax
from jax.experimental import pallas as pl

N = 50000
C = 128
NCOEF = 25
NA = 100
BT = 400


def _emb_fill(idx_ref, table_ref, o_ref):
    onehot = (idx_ref[...] == lax.broadcasted_iota(jnp.int32, (BT, NA), 1))
    g = lax.dot(onehot.astype(jnp.float32), table_ref[...],
                precision=lax.Precision.HIGHEST,
                preferred_element_type=jnp.float32)
    coef = lax.broadcasted_iota(jnp.int32, (BT, NCOEF, C), 1)
    o_ref[...] = jnp.where(coef == 0, g[:, None, :], 0.0)


def kernel(atomic_numbers, embedding_table):
    idx = atomic_numbers.astype(jnp.int32).reshape(N, 1)
    return pl.pallas_call(
        _emb_fill,
        grid=(N // BT,),
        in_specs=[
            pl.BlockSpec((BT, 1), lambda i: (i, 0)),
            pl.BlockSpec((NA, C), lambda i: (0, 0)),
        ],
        out_specs=pl.BlockSpec((BT, NCOEF, C), lambda i: (i, 0, 0)),
        out_shape=jax.ShapeDtypeStruct((N, NCOEF, C), jnp.float32),
    )(idx, embedding_table)


# speedup vs baseline: 1.6001x; 1.0007x over previous
import jax
import jax.numpy as jnp
from jax import lax
from jax.experimental import pallas as pl
from jax.experimental.pallas import tpu as pltpu

N = 50000
C = 128
NCOEF = 25
NA = 100
BT = 400


def _emb_fill(idx_ref, table_ref, o_ref):
    onehot = (idx_ref[...] == lax.broadcasted_iota(jnp.int32, (BT, NA), 1))
    g = lax.dot(onehot.astype(jnp.float32), table_ref[...],
                precision=lax.Precision.HIGHEST,
                preferred_element_type=jnp.float32)
    coef = lax.broadcasted_iota(jnp.int32, (BT, NCOEF, C), 1)
    o_ref[...] = jnp.where(coef == 0, g[:, None, :], 0.0)


def kernel(atomic_numbers, embedding_table):
    idx = atomic_numbers.astype(jnp.int32).reshape(N, 1)
    return pl.pallas_call(
        _emb_fill,
        grid=(N // BT,),
        in_specs=[
            pl.BlockSpec((BT, 1), lambda i: (i, 0)),
            pl.BlockSpec((NA, C), lambda i: (0, 0)),
        ],
        out_specs=pl.BlockSpec((BT, NCOEF, C), lambda i: (i, 0, 0)),
        out_shape=jax.ShapeDtypeStruct((N, NCOEF, C), jnp.float32),
        compiler_params=pltpu.CompilerParams(
            dimension_semantics=("parallel",)),
    )(idx, embedding_table)


# speedup vs baseline: 1.6084x; 1.0052x over previous
import jax
import jax.numpy as jnp
from jax import lax
from jax.experimental import pallas as pl
from jax.experimental.pallas import tpu as pltpu

N = 50000
C = 128
NCOEF = 25
NA = 100
BT = 1000


def _emb_fill(idx_ref, table_ref, o_ref):
    onehot = (idx_ref[...] == lax.broadcasted_iota(jnp.int32, (BT, NA), 1))
    g = lax.dot(onehot.astype(jnp.float32), table_ref[...],
                precision=lax.Precision.HIGHEST,
                preferred_element_type=jnp.float32)
    coef = lax.broadcasted_iota(jnp.int32, (BT, NCOEF, C), 1)
    o_ref[...] = jnp.where(coef == 0, g[:, None, :], 0.0)


def kernel(atomic_numbers, embedding_table):
    idx = atomic_numbers.astype(jnp.int32).reshape(N, 1)
    return pl.pallas_call(
        _emb_fill,
        grid=(N // BT,),
        in_specs=[
            pl.BlockSpec((BT, 1), lambda i: (i, 0)),
            pl.BlockSpec((NA, C), lambda i: (0, 0)),
        ],
        out_specs=pl.BlockSpec((BT, NCOEF, C), lambda i: (i, 0, 0)),
        out_shape=jax.ShapeDtypeStruct((N, NCOEF, C), jnp.float32),
        compiler_params=pltpu.CompilerParams(
            dimension_semantics=("parallel",)),
    )(idx, embedding_table)


# speedup vs baseline: 1.7050x; 1.0600x over previous
import functools

import jax
import jax.numpy as jnp
from jax import lax
from jax.experimental import pallas as pl
from jax.experimental.pallas import tpu as pltpu
from jax.experimental.pallas import tpu_sc as plsc

N = 50000
C = 128
NCOEF = 25
NZ = NCOEF - 1
NW = 32
S = 1584
R = 8
CH = S // R
NB = 3
GROUPS = CH // NB
LA = 1

_mesh = plsc.VectorSubcoreMesh(core_axis_name="c", subcore_axis_name="s")


@functools.partial(
    pl.kernel,
    mesh=_mesh,
    out_type=jax.ShapeDtypeStruct((N, NCOEF, C), jnp.float32),
    scratch_types=[
        pltpu.VMEM((S,), jnp.int32),
        pltpu.VMEM((NB, R, NCOEF, C), jnp.float32),
        pltpu.VMEM_SHARED((100, 1, C), jnp.float32),
        pltpu.SemaphoreType.DMA,
        pltpu.SemaphoreType.DMA,
        pltpu.SemaphoreType.DMA,
        pltpu.SemaphoreType.DMA,
        pltpu.SemaphoreType.DMA,
        pltpu.SemaphoreType.DMA,
    ],
)
def _emb_sc(idx_hbm, table_hbm, zeros_hbm, out_hbm, idx_v, buf, table_s,
            gsem0, gsem1, gsem2, wsem0, wsem1, wsem2):
    cid = lax.axis_index("c")
    sid = lax.axis_index("s")
    wid = sid * 2 + cid
    gsem = (gsem0, gsem1, gsem2)
    wsem = (wsem0, wsem1, wsem2)
    base_w = jnp.minimum(wid * S, N - S)

    for b in range(NB):
        pltpu.sync_copy(zeros_hbm, buf.at[b, :, pl.ds(1, NZ), :])

    pltpu.sync_copy(idx_hbm.at[pl.ds(base_w, S)], idx_v)

    @pl.when(sid == 0)
    def _():
        pltpu.sync_copy(table_hbm, table_s)

    plsc.subcore_barrier()

    for b in range(LA):
        pltpu.async_copy(
            table_s.at[idx_v.at[pl.ds(b * R, R)]],
            buf.at[b, :, pl.ds(0, 1), :], gsem[b],
        )

    def group(g, carry):
        for b in range(NB):
            c = NB * g + b
            pltpu.make_async_copy(
                table_s.at[idx_v.at[pl.ds(0, R)]],
                buf.at[b, :, pl.ds(0, 1), :], gsem[b],
            ).wait()
            pltpu.async_copy(
                buf.at[b], out_hbm.at[pl.ds(base_w + c * R, R)], wsem[b]
            )

            bn = (b + LA) % NB

            @pl.when(jnp.logical_and(c >= NB - LA, c + LA < CH))
            def _():
                pltpu.make_async_copy(
                    buf.at[bn], out_hbm.at[pl.ds(0, R)], wsem[bn]
                ).wait()

            @pl.when(c + LA < CH)
            def _():
                pltpu.async_copy(
                    table_s.at[idx_v.at[pl.ds((c + LA) * R, R)]],
                    buf.at[bn, :, pl.ds(0, 1), :], gsem[bn],
                )
        return carry

    lax.fori_loop(0, GROUPS, group, 0)

    for b in range(NB):
        pltpu.make_async_copy(
            buf.at[b], out_hbm.at[pl.ds(0, R)], wsem[b]
        ).wait()


def kernel(atomic_numbers, embedding_table):
    idx = atomic_numbers.astype(jnp.int32)
    table3 = embedding_table.reshape(embedding_table.shape[0], 1, C)
    zeros = jnp.zeros((R, NZ, C), jnp.float32)
    return _emb_sc(idx, table3, zeros)
